# Initial kernel scaffold; baseline (speedup 1.0000x reference)
import jax
import jax.numpy as jnp
from jax.experimental import pallas as pl


def kernel(x, neighbors, params):
    raise NotImplementedError("write your pallas kernel here")


# speedup vs baseline: 1.3810x; 1.3810x over previous
import functools

import jax
import jax.numpy as jnp
from jax import lax
from jax.experimental import pallas as pl
from jax.experimental.pallas import tpu as pltpu
from jax.experimental.pallas import tpu_sc as plsc

_N = 10000
_K = 6
_NW = 32
_PW = 320
_NP = _NW * _PW
_S = 80
_BN = 400


def _sc_gather(h, nbt):
    C = h.shape[1]
    mesh = plsc.VectorSubcoreMesh(core_axis_name="c", subcore_axis_name="s")
    nc = mesh.num_cores

    def body(h_hbm, nbt_hbm, g_hbm, idx_v, buf_v, sem):
        wid = lax.axis_index("s") * nc + lax.axis_index("c")
        base = wid * _PW

        def step(j, carry):
            row = base + j * _S
            for k in range(_K):
                pltpu.sync_copy(nbt_hbm.at[pl.ds(k * _NP + row, _S)], idx_v)
                pltpu.async_copy(h_hbm.at[idx_v], buf_v, sem).wait()
                pltpu.sync_copy(buf_v, g_hbm.at[k, pl.ds(row, _S)])
            return carry

        lax.fori_loop(0, _PW // _S, step, 0)

    f = pl.kernel(
        body,
        out_type=jax.ShapeDtypeStruct((_K, _NP, C), h.dtype),
        mesh=mesh,
        scratch_types=[
            pltpu.VMEM((_S,), jnp.int32),
            pltpu.VMEM((_S, C), h.dtype),
            pltpu.SemaphoreType.DMA,
        ],
    )
    return f(h, nbt)


def _tc_conv(parts, out_ch):
    nparts = len(parts)

    def body(*refs):
        o_ref = refs[-1]
        acc = None
        for p in range(nparts):
            h_ref, g_ref, w_ref = refs[3 * p : 3 * p + 3]
            t = jnp.dot(h_ref[...], w_ref[0], preferred_element_type=jnp.float32)
            for k in range(_K):
                t = t + jnp.dot(g_ref[k], w_ref[k + 1],
                                preferred_element_type=jnp.float32)
            acc = t if acc is None else acc + t
        o_ref[...] = acc

    in_specs = []
    args = []
    for (h, g, wt) in parts:
        C = h.shape[1]
        in_specs.append(pl.BlockSpec((_BN, C), lambda i: (i, 0)))
        in_specs.append(pl.BlockSpec((_K, _BN, C), lambda i: (0, i, 0)))
        in_specs.append(pl.BlockSpec((_K + 1, C, out_ch), lambda i: (0, 0, 0)))
        args += [h, g, wt]

    return pl.pallas_call(
        body,
        grid=(_N // _BN,),
        in_specs=in_specs,
        out_specs=pl.BlockSpec((_BN, out_ch), lambda i: (i, 0)),
        out_shape=jax.ShapeDtypeStruct((_N, out_ch), jnp.float32),
    )(*args)


def _norm_act(raw, res=None):
    n, c = raw.shape

    def body_plain(x_ref, o_ref):
        x = x_ref[...]
        m = jnp.mean(x, axis=0, keepdims=True)
        v = jnp.mean(jnp.square(x - m), axis=0, keepdims=True)
        o_ref[...] = jnp.maximum((x - m) * lax.rsqrt(v + 1e-5), 0.0)

    def body_res(x_ref, r_ref, o_ref):
        x = x_ref[...]
        m = jnp.mean(x, axis=0, keepdims=True)
        v = jnp.mean(jnp.square(x - m), axis=0, keepdims=True)
        o_ref[...] = jnp.maximum((x - m) * lax.rsqrt(v + 1e-5) + r_ref[...], 0.0)

    bc = 128
    out_shape = jax.ShapeDtypeStruct((n, c), jnp.float32)
    spec = pl.BlockSpec((n, bc), lambda j: (0, j))
    if res is None:
        return pl.pallas_call(body_plain, grid=(c // bc,), in_specs=[spec],
                              out_specs=spec, out_shape=out_shape)(raw)
    return pl.pallas_call(body_res, grid=(c // bc,), in_specs=[spec, spec],
                          out_specs=spec, out_shape=out_shape)(raw, res)


def kernel(x, neighbors, params):
    h0 = x[0].T
    nbt = jnp.zeros((_K, _NP), jnp.int32)
    nbt = nbt.at[:, :_N].set(neighbors.astype(jnp.int32).T).reshape(-1)

    wts = [jnp.transpose(w, (2, 1, 0)) for (w, _) in params]

    def gather(h):
        return _sc_gather(h, nbt)

    pi = 0
    h = h0
    hg = gather(h)
    enc = []
    for _ in range(3):
        o1 = wts[pi].shape[2]
        a = _norm_act(_tc_conv([(h, hg, wts[pi])], o1)); pi += 1
        ag = gather(a)
        r = _tc_conv([(a, ag, wts[pi])], o1); pi += 1
        h = _norm_act(r, res=a)
        hg = gather(h)
        enc.append((h, hg))

    for i in range(2):
        ou = wts[pi].shape[2]
        u = _tc_conv([(h, hg, wts[pi])], ou); pi += 1
        ug = gather(u)
        eh, eg = enc[1 - i]
        cu = u.shape[1]
        wcat = wts[pi]; pi += 1
        o1 = wcat.shape[2]
        a = _norm_act(
            _tc_conv([(u, ug, wcat[:, :cu, :]), (eh, eg, wcat[:, cu:, :])], o1))
        ag = gather(a)
        r = _tc_conv([(a, ag, wts[pi])], o1); pi += 1
        h = _norm_act(r, res=a)
        hg = gather(h)

    ou = wts[pi].shape[2]
    u = _tc_conv([(h, hg, wts[pi])], ou); pi += 1
    ug = gather(u)
    o1 = wts[pi].shape[2]
    a = _norm_act(_tc_conv([(u, ug, wts[pi])], o1)); pi += 1
    ag = gather(a)
    r = _tc_conv([(a, ag, wts[pi])], o1); pi += 1
    out = _norm_act(r, res=a)

    return out.T[None]
